# Initial kernel scaffold; baseline (speedup 1.0000x reference)
import jax
import jax.numpy as jnp
from jax.experimental import pallas as pl


def kernel(input, semantic_label, Wx, Wy, Wout, bout):
    raise NotImplementedError("write your pallas kernel here")


# speedup vs baseline: 1316.7742x; 1316.7742x over previous
import functools

import jax
import jax.numpy as jnp
from jax import lax
from jax.experimental import pallas as pl
from jax.experimental.pallas import tpu as pltpu

CH = 384
WIN = 64
NS = 4
RED = 4
CX = CH // RED


def _pick_div(n, prefs):
    for p in prefs:
        if n % p == 0:
            return p
    return 1


def _conv_body(a_ref, b_ref, wx_ref, wy_ref, x_ref, y_ref, *, tile, wp):
    win = jnp.concatenate([a_ref[...], b_ref[...]], axis=1)
    acc = jnp.zeros((tile, CX), dtype=jnp.float32)
    for di in range(3):
        for dj in range(3):
            off = di * wp + dj
            acc = acc + lax.dot_general(
                win[:, off:off + tile], wx_ref[di * 3 + dj],
                (((0,), (0,)), ((), ())),
                preferred_element_type=jnp.float32)
    x_ref[...] = acc
    y_ref[...] = lax.dot_general(
        win[:, wp + 1:wp + 1 + tile], wy_ref[...],
        (((0,), (0,)), ((), ())),
        preferred_element_type=jnp.float32)


def _convs(inp, wx_t, wy_t, hh, ww):
    wp = ww + 2
    inp_p = jnp.pad(inp[None], ((0, 0), (0, 0), (1, 1), (1, 1)),
                    mode='reflect')[0]
    flat = inp_p.reshape(CH, (hh + 2) * wp)
    lp = hh * wp
    tile = 1024
    grid = -(-lp // tile)
    need = (grid + 1) * tile
    flat = jnp.pad(flat, ((0, 0), (0, need - flat.shape[1])))

    x_pad, y_pad = pl.pallas_call(
        functools.partial(_conv_body, tile=tile, wp=wp),
        grid=(grid,),
        in_specs=[
            pl.BlockSpec((CH, tile), lambda i: (0, i)),
            pl.BlockSpec((CH, tile), lambda i: (0, i)),
            pl.BlockSpec((9, CH, CX), lambda i: (0, 0, 0)),
            pl.BlockSpec((CH, CH), lambda i: (0, 0)),
        ],
        out_specs=[
            pl.BlockSpec((tile, CX), lambda i: (i, 0)),
            pl.BlockSpec((tile, CH), lambda i: (i, 0)),
        ],
        out_shape=[
            jax.ShapeDtypeStruct((grid * tile, CX), jnp.float32),
            jax.ShapeDtypeStruct((grid * tile, CH), jnp.float32),
        ],
    )(flat[:, :grid * tile], flat[:, tile:], wx_t, wy_t)
    return x_pad, y_pad


def _attn_body(xc_ref, xp_ref, xn_ref, yc_ref, yp_ref, yn_ref, o_ref, *, cpb):
    for c in range(cpb):
        q = xc_ref[0, c * WIN:(c + 1) * WIN, :]
        if c == 0:
            xprev = xp_ref[0]
        else:
            xprev = xc_ref[0, (c - 1) * WIN:c * WIN, :]
        if c == cpb - 1:
            xnext = xn_ref[0]
        else:
            xnext = xc_ref[0, (c + 1) * WIN:(c + 2) * WIN, :]
        k = jnp.concatenate([q, xprev, xnext], axis=0)
        norm = jnp.sqrt(jnp.sum(k * k, axis=1, keepdims=True))
        k = k / jnp.maximum(norm, 5e-5)
        raw = lax.dot_general(q, k, (((1,), (1,)), ((), ())),
                              preferred_element_type=jnp.float32)
        raw = raw - jnp.max(raw, axis=1, keepdims=True)
        w = jnp.exp(raw)
        w = w / jnp.sum(w, axis=1, keepdims=True)
        if c == 0:
            yprev = yp_ref[0]
        else:
            yprev = yc_ref[0, (c - 1) * WIN:c * WIN, :]
        if c == cpb - 1:
            ynext = yn_ref[0]
        else:
            ynext = yc_ref[0, (c + 1) * WIN:(c + 2) * WIN, :]
        v = jnp.concatenate(
            [yc_ref[0, c * WIN:(c + 1) * WIN, :], yprev, ynext], axis=0)
        o_ref[0, c * WIN:(c + 1) * WIN, :] = jnp.dot(
            w, v, preferred_element_type=jnp.float32)


def _attention(xs, ys, nchunk):
    cpb = _pick_div(nchunk, (8, 7, 4, 2))
    nblk = nchunk // cpb
    bt = cpb * WIN
    ll = xs.shape[1]

    out = pl.pallas_call(
        functools.partial(_attn_body, cpb=cpb),
        grid=(NS, nblk),
        in_specs=[
            pl.BlockSpec((1, bt, CX), lambda s, b: (s, b, 0)),
            pl.BlockSpec((1, WIN, CX),
                         lambda s, b: (s, (b * cpb - 1) % nchunk, 0)),
            pl.BlockSpec((1, WIN, CX),
                         lambda s, b: (s, (b * cpb + cpb) % nchunk, 0)),
            pl.BlockSpec((1, bt, CH), lambda s, b: (s, b, 0)),
            pl.BlockSpec((1, WIN, CH),
                         lambda s, b: (s, (b * cpb - 1) % nchunk, 0)),
            pl.BlockSpec((1, WIN, CH),
                         lambda s, b: (s, (b * cpb + cpb) % nchunk, 0)),
        ],
        out_specs=pl.BlockSpec((1, bt, CH), lambda s, b: (s, b, 0)),
        out_shape=jax.ShapeDtypeStruct((NS, ll, CH), jnp.float32),
    )(xs, xs, xs, ys, ys, ys)
    return out


def _out_body(a_ref, w_ref, b_ref, r_ref, o_ref):
    acc = lax.dot_general(a_ref[...], w_ref[...], (((1,), (0,)), ((), ())),
                          preferred_element_type=jnp.float32)
    o_ref[...] = acc + b_ref[...] + r_ref[...]


def _out_proj(amat, wout_r, bout, inp_t, ll):
    t3 = _pick_div(ll, (512, 448, 256, 128, 64, 8))
    grid = ll // t3
    return pl.pallas_call(
        _out_body,
        grid=(grid,),
        in_specs=[
            pl.BlockSpec((t3, NS * CH), lambda i: (i, 0)),
            pl.BlockSpec((NS * CH, CH), lambda i: (0, 0)),
            pl.BlockSpec((1, CH), lambda i: (0, 0)),
            pl.BlockSpec((t3, CH), lambda i: (i, 0)),
        ],
        out_specs=pl.BlockSpec((t3, CH), lambda i: (i, 0)),
        out_shape=jax.ShapeDtypeStruct((ll, CH), jnp.float32),
    )(amat, wout_r, bout, inp_t)


def kernel(input, semantic_label, Wx, Wy, Wout, bout):
    n, _, hh, ww = input.shape
    ll = hh * ww
    wp = ww + 2
    nchunk = ll // WIN

    inp = input[0]
    wx_t = jnp.transpose(Wx, (2, 3, 1, 0)).reshape(9, CH, CX)
    wy_t = Wy[:, :, 0, 0].T
    wout_r = Wout[:, :, 0, 0].T

    x_pad, y_pad = _convs(inp, wx_t, wy_t, hh, ww)

    sl = semantic_label.reshape(NS, ll)
    offsets = (jnp.arange(NS) * 100).reshape(-1, 1)
    keys = (sl + offsets).reshape(-1)
    indices = jnp.argsort(keys)
    undo_sort = jnp.argsort(indices)
    mod_idx = indices % ll
    idx_pad = (mod_idx // ww) * wp + (mod_idx % ww)

    xs = jnp.take(x_pad, idx_pad, axis=0).reshape(NS, ll, CX)
    ys = jnp.take(y_pad, idx_pad, axis=0).reshape(NS, ll, CH)

    att = _attention(xs, ys, nchunk)

    idx2 = undo_sort.reshape(NS, ll).T.reshape(-1)
    amat = jnp.take(att.reshape(NS * ll, CH), idx2, axis=0)
    amat = amat.reshape(ll, NS * CH)

    inp_t = inp.reshape(CH, ll).T
    out = _out_proj(amat, wout_r, bout.reshape(1, CH), inp_t, ll)
    return out.T.reshape(n, CH, hh, ww)
